# Initial kernel scaffold; baseline (speedup 1.0000x reference)
import jax
import jax.numpy as jnp
from jax.experimental import pallas as pl


def kernel(x, adj, e, W):
    raise NotImplementedError("write your pallas kernel here")


# speedup vs baseline: 14.2882x; 14.2882x over previous
import functools

import jax
import jax.numpy as jnp
from jax import lax
from jax.experimental import pallas as pl
from jax.experimental.pallas import tpu as pltpu
from jax.experimental.pallas import tpu_sc as plsc

N = 10000
N_PAD = 10240
E = 320000
D = 128
K = 4
NC = 2
NS = 16
NW = NC * NS
CH = 128
ZROWS = 128
NCHUNKS = E // CH
ROWS_PER_TILE = N_PAD // NS
DEG_L = 16

@functools.lru_cache(maxsize=None)
def _sc_mesh():
    return plsc.VectorSubcoreMesh(
        core_axis_name="c", subcore_axis_name="s",
        num_cores=NC, num_subcores=NS)


def _chunk_count(wid):
    rem = NCHUNKS % NW
    return NCHUNKS // NW + jnp.where(wid < rem, 1, 0).astype(jnp.int32)


def _deg_body(col, out, acc, ones_v, idx_v, zbuf):
    cid = lax.axis_index("c")
    sid = lax.axis_index("s")
    wid = cid * NS + sid

    zero16 = jnp.zeros((16,), jnp.float32)
    one16 = jnp.ones((16,), jnp.float32)

    def zbody(i, _):
        for j in range(D // 16):
            zbuf[i, pl.ds(j * 16, 16)] = zero16
        return 0
    lax.fori_loop(0, ZROWS, zbody, 0)
    for p in range(ROWS_PER_TILE // ZROWS):
        pltpu.sync_copy(
            zbuf, acc.at[pl.ds(sid * ROWS_PER_TILE + p * ZROWS, ZROWS)])

    def obody(i, _):
        for j in range(D // 16):
            ones_v[i, pl.ds(j * 16, 16)] = one16
        return 0
    lax.fori_loop(0, CH, obody, 0)

    plsc.subcore_barrier()

    def ebody(i, _):
        c = wid + i * NW
        pltpu.sync_copy(col.at[pl.ds(c * CH, CH)], idx_v)
        pltpu.sync_copy(ones_v, acc.at[idx_v], add=True)
        return 0
    lax.fori_loop(0, _chunk_count(wid), ebody, 0)

    plsc.subcore_barrier()
    pltpu.sync_copy(acc.at[pl.ds(sid * ROWS_PER_TILE, ROWS_PER_TILE)],
                    out.at[cid, pl.ds(sid * ROWS_PER_TILE, ROWS_PER_TILE)])


@functools.lru_cache(maxsize=None)
def _deg_call():
  return pl.kernel(
    _deg_body,
    out_type=jax.ShapeDtypeStruct((NC, N_PAD, D), jnp.float32),
    mesh=_sc_mesh(),
    scratch_types=[
        pltpu.VMEM_SHARED((N_PAD, D), jnp.float32),
        pltpu.VMEM((CH, D), jnp.float32),
        pltpu.VMEM((CH,), jnp.int32),
        pltpu.VMEM((ZROWS, D), jnp.float32),
    ],
  )


def _gs_body(xs, row, col, out, acc, ridx, cidx, gbuf, zbuf, sem):
    cid = lax.axis_index("c")
    sid = lax.axis_index("s")
    wid = cid * NS + sid

    zero16 = jnp.zeros((16,), jnp.float32)

    def zbody(i, _):
        for j in range(D // 16):
            zbuf[i, pl.ds(j * 16, 16)] = zero16
        return 0
    lax.fori_loop(0, ZROWS, zbody, 0)
    for p in range(ROWS_PER_TILE // ZROWS):
        pltpu.sync_copy(
            zbuf, acc.at[pl.ds(sid * ROWS_PER_TILE + p * ZROWS, ZROWS)])

    plsc.subcore_barrier()

    def ebody(i, _):
        c = wid + i * NW
        pltpu.sync_copy(row.at[pl.ds(c * CH, CH)], ridx)
        pltpu.sync_copy(col.at[pl.ds(c * CH, CH)], cidx)
        pltpu.async_copy(xs.at[ridx], gbuf, sem).wait()
        pltpu.sync_copy(gbuf, acc.at[cidx], add=True)
        return 0
    lax.fori_loop(0, _chunk_count(wid), ebody, 0)

    plsc.subcore_barrier()
    pltpu.sync_copy(acc.at[pl.ds(sid * ROWS_PER_TILE, ROWS_PER_TILE)],
                    out.at[cid, pl.ds(sid * ROWS_PER_TILE, ROWS_PER_TILE)])


@functools.lru_cache(maxsize=None)
def _gs_call():
  return pl.kernel(
    _gs_body,
    out_type=jax.ShapeDtypeStruct((NC, N_PAD, D), jnp.float32),
    mesh=_sc_mesh(),
    scratch_types=[
        pltpu.VMEM_SHARED((N_PAD, D), jnp.float32),
        pltpu.VMEM((CH,), jnp.int32),
        pltpu.VMEM((CH,), jnp.int32),
        pltpu.VMEM((CH, D), jnp.float32),
        pltpu.VMEM((ZROWS, D), jnp.float32),
        pltpu.SemaphoreType.DMA,
    ],
  )


BN = 1024


def _deg_col(dp_ref):
    return dp_ref[0, :, 0:1] + dp_ref[1, :, 0:1]


def _xs_body(dp_ref, x_ref, o_ref):
    deg = _deg_col(dp_ref)
    r = jnp.where(deg > 0, lax.rsqrt(deg), 0.0)
    o_ref[...] = r * x_ref[...]


_xs_call = pl.pallas_call(
    _xs_body,
    out_shape=jax.ShapeDtypeStruct((N, D), jnp.float32),
    grid=((N + BN - 1) // BN,),
    in_specs=[
        pl.BlockSpec((NC, BN, D), lambda i: (0, i, 0)),
        pl.BlockSpec((BN, D), lambda i: (i, 0)),
    ],
    out_specs=pl.BlockSpec((BN, D), lambda i: (i, 0)),
)


def _comb_body(dp_ref, hp_ref, x_ref, e_ref, w_ref, o_ref):
    deg = _deg_col(dp_ref)
    r = jnp.where(deg > 0, lax.rsqrt(deg), 0.0)
    hi = r * (hp_ref[0] + hp_ref[1])
    xb = x_ref[...]
    hcat = jnp.concatenate([hi, xb], axis=1)
    y = jnp.dot(hcat, w_ref[...], preferred_element_type=jnp.float32)
    acc = xb
    eb = e_ref[...]
    for k in range(K):
        acc = acc + eb[:, k][:, None] * y[:, k * D:(k + 1) * D]
    o_ref[...] = acc


_comb_call = pl.pallas_call(
    _comb_body,
    out_shape=jax.ShapeDtypeStruct((N, D), jnp.float32),
    grid=((N + BN - 1) // BN,),
    in_specs=[
        pl.BlockSpec((NC, BN, D), lambda i: (0, i, 0)),
        pl.BlockSpec((NC, BN, D), lambda i: (0, i, 0)),
        pl.BlockSpec((BN, D), lambda i: (i, 0)),
        pl.BlockSpec((BN, K), lambda i: (i, 0)),
        pl.BlockSpec((2 * D, K * D), lambda i: (0, 0)),
    ],
    out_specs=pl.BlockSpec((BN, D), lambda i: (i, 0)),
)


def kernel(x, adj, e, W):
    row = adj[0]
    col = adj[1]
    deg_part = _deg_call()(col)
    xs = _xs_call(deg_part, x)
    hi_part = _gs_call()(xs, row, col)
    Wr = jnp.transpose(W, (1, 0, 2)).reshape(2 * D, K * D)
    return _comb_call(deg_part, hi_part, x, e, Wr)


# speedup vs baseline: 21.0570x; 1.4737x over previous
import functools

import jax
import jax.numpy as jnp
from jax import lax
from jax.experimental import pallas as pl
from jax.experimental.pallas import tpu as pltpu
from jax.experimental.pallas import tpu_sc as plsc

N = 10000
N_PAD = 10240
E = 320000
D = 128
K = 4
NC = 2
NS = 16
NW = NC * NS
CH = 128
NCH_T = 80
NCH_TOT = NCH_T * NW
E_PAD = NCH_TOT * CH
ROWS_PER_TILE = N_PAD // NS
NBUF = 2
NHALF = 2
CPH = NCH_T // NHALF
NGRP = CPH // NBUF
ZROWS = 32


@functools.lru_cache(maxsize=None)
def _sc_mesh():
    return plsc.VectorSubcoreMesh(
        core_axis_name="c", subcore_axis_name="s",
        num_cores=NC, num_subcores=NS)


def _zero_acc_slice(acc, zbuf, zr, sid):
    zero16 = jnp.zeros((16,), jnp.float32)

    def zbody(i, _):
        for j in range(D // 16):
            zbuf[i, pl.ds(j * 16, 16)] = zero16
        return 0
    lax.fori_loop(0, zr, zbody, 0)
    for p in range(ROWS_PER_TILE // zr):
        pltpu.sync_copy(
            zbuf, acc.at[pl.ds(sid * ROWS_PER_TILE + p * zr, zr)])


def _deg_body(col2, out, acc, ones_v, cidx, zbuf, s0, s1):
    cid = lax.axis_index("c")
    sid = lax.axis_index("s")
    wid = cid * NS + sid
    ssem = (s0, s1)

    _zero_acc_slice(acc, zbuf, ZROWS, sid)

    one16 = jnp.ones((16,), jnp.float32)

    def obody(i, _):
        for j in range(D // 16):
            ones_v[i, pl.ds(j * 16, 16)] = one16
        return 0
    lax.fori_loop(0, CH, obody, 0)
    plsc.subcore_barrier()

    for h in range(NHALF):
        pltpu.sync_copy(
            col2.at[pl.ds(wid * NCH_T + h * CPH, CPH)], cidx)

        def grp(g, _):
            for b in range(NBUF):
                @pl.when(g > 0)
                def _wait(b=b, g=g):
                    pltpu.make_async_copy(
                        ones_v, acc.at[cidx.at[(g - 1) * NBUF + b]],
                        ssem[b]).wait()
                pltpu.async_copy(ones_v, acc.at[cidx.at[g * NBUF + b]],
                                 ssem[b], add=True)
            return 0
        lax.fori_loop(0, NGRP, grp, 0)
        for b in range(NBUF):
            pltpu.make_async_copy(
                ones_v, acc.at[cidx.at[(NGRP - 1) * NBUF + b]],
                ssem[b]).wait()

    plsc.subcore_barrier()
    pltpu.sync_copy(acc.at[pl.ds(sid * ROWS_PER_TILE, ROWS_PER_TILE)],
                    out.at[cid, pl.ds(sid * ROWS_PER_TILE, ROWS_PER_TILE)])


@functools.lru_cache(maxsize=None)
def _deg_call():
  return pl.kernel(
    _deg_body,
    out_type=jax.ShapeDtypeStruct((NC, N_PAD, D), jnp.float32),
    mesh=_sc_mesh(),
    scratch_types=[
        pltpu.VMEM_SHARED((N_PAD, D), jnp.float32),
        pltpu.VMEM((CH, D), jnp.float32),
        pltpu.VMEM((CPH, CH), jnp.int32),
        pltpu.VMEM((ZROWS, D), jnp.float32),
        pltpu.SemaphoreType.DMA,
        pltpu.SemaphoreType.DMA,
    ],
  )


def _gs_body(xs, row2, col2, out, acc, ridx, cidx, gbuf,
             g0, g1, s0, s1):
    cid = lax.axis_index("c")
    sid = lax.axis_index("s")
    wid = cid * NS + sid
    gsem = (g0, g1)
    ssem = (s0, s1)

    _zero_acc_slice(acc, gbuf.at[0], CH, sid)
    plsc.subcore_barrier()

    for h in range(NHALF):
        pltpu.sync_copy(
            row2.at[pl.ds(wid * NCH_T + h * CPH, CPH)], ridx)
        pltpu.sync_copy(
            col2.at[pl.ds(wid * NCH_T + h * CPH, CPH)], cidx)

        for b in range(NBUF):
            pltpu.async_copy(xs.at[ridx.at[b]], gbuf.at[b], gsem[b])

        def grp(g, _):
            base = g * NBUF
            for b in range(NBUF):
                pltpu.make_async_copy(
                    xs.at[ridx.at[base + b]], gbuf.at[b], gsem[b]).wait()
                pltpu.async_copy(gbuf.at[b], acc.at[cidx.at[base + b]],
                                 ssem[b], add=True)

            @pl.when(g + 1 < NGRP)
            def _prefetch(g=g, base=base):
                for b in range(NBUF):
                    pltpu.make_async_copy(
                        gbuf.at[b], acc.at[cidx.at[base + b]],
                        ssem[b]).wait()
                    pltpu.async_copy(xs.at[ridx.at[base + NBUF + b]],
                                     gbuf.at[b], gsem[b])
            return 0
        lax.fori_loop(0, NGRP, grp, 0)
        for b in range(NBUF):
            pltpu.make_async_copy(
                gbuf.at[b], acc.at[cidx.at[(NGRP - 1) * NBUF + b]],
                ssem[b]).wait()

    plsc.subcore_barrier()
    pltpu.sync_copy(acc.at[pl.ds(sid * ROWS_PER_TILE, ROWS_PER_TILE)],
                    out.at[cid, pl.ds(sid * ROWS_PER_TILE, ROWS_PER_TILE)])


@functools.lru_cache(maxsize=None)
def _gs_call():
  return pl.kernel(
    _gs_body,
    out_type=jax.ShapeDtypeStruct((NC, N_PAD, D), jnp.float32),
    mesh=_sc_mesh(),
    scratch_types=[
        pltpu.VMEM_SHARED((N_PAD, D), jnp.float32),
        pltpu.VMEM((CPH, CH), jnp.int32),
        pltpu.VMEM((CPH, CH), jnp.int32),
        pltpu.VMEM((NBUF, CH, D), jnp.float32),
        pltpu.SemaphoreType.DMA,
        pltpu.SemaphoreType.DMA,
        pltpu.SemaphoreType.DMA,
        pltpu.SemaphoreType.DMA,
    ],
  )


BN = 1024


def _deg_col(dp_ref):
    return dp_ref[0, :, 0:1] + dp_ref[1, :, 0:1]


def _xs_body(dp_ref, x_ref, o_ref):
    deg = _deg_col(dp_ref)
    r = jnp.where(deg > 0, lax.rsqrt(deg), 0.0)
    o_ref[...] = jnp.where(deg > 0, r * x_ref[...], 0.0)


_xs_call = pl.pallas_call(
    _xs_body,
    out_shape=jax.ShapeDtypeStruct((N_PAD, D), jnp.float32),
    grid=(N_PAD // BN,),
    in_specs=[
        pl.BlockSpec((NC, BN, D), lambda i: (0, i, 0)),
        pl.BlockSpec((BN, D), lambda i: (i, 0)),
    ],
    out_specs=pl.BlockSpec((BN, D), lambda i: (i, 0)),
)


def _comb_body(dp_ref, hp_ref, x_ref, e_ref, w_ref, o_ref):
    deg = _deg_col(dp_ref)
    r = jnp.where(deg > 0, lax.rsqrt(deg), 0.0)
    hi = r * (hp_ref[0] + hp_ref[1])
    xb = x_ref[...]
    hcat = jnp.concatenate([hi, xb], axis=1)
    y = jnp.dot(hcat, w_ref[...], preferred_element_type=jnp.float32)
    acc = xb
    eb = e_ref[...]
    for k in range(K):
        acc = acc + eb[:, k][:, None] * y[:, k * D:(k + 1) * D]
    o_ref[...] = acc


_comb_call = pl.pallas_call(
    _comb_body,
    out_shape=jax.ShapeDtypeStruct((N, D), jnp.float32),
    grid=((N + BN - 1) // BN,),
    in_specs=[
        pl.BlockSpec((NC, BN, D), lambda i: (0, i, 0)),
        pl.BlockSpec((NC, BN, D), lambda i: (0, i, 0)),
        pl.BlockSpec((BN, D), lambda i: (i, 0)),
        pl.BlockSpec((BN, K), lambda i: (i, 0)),
        pl.BlockSpec((2 * D, K * D), lambda i: (0, 0)),
    ],
    out_specs=pl.BlockSpec((BN, D), lambda i: (i, 0)),
)


def kernel(x, adj, e, W):
    pad_idx = N + (jnp.arange(E_PAD - E, dtype=jnp.int32) % (N_PAD - N))
    row2 = jnp.concatenate([adj[0], pad_idx]).reshape(NCH_TOT, CH)
    col2 = jnp.concatenate([adj[1], pad_idx]).reshape(NCH_TOT, CH)
    deg_part = _deg_call()(col2)
    xs = _xs_call(deg_part, x)
    hi_part = _gs_call()(xs, row2, col2)
    Wr = jnp.transpose(W, (1, 0, 2)).reshape(2 * D, K * D)
    return _comb_call(deg_part, hi_part, x, e, Wr)


# speedup vs baseline: 23.1551x; 1.0996x over previous
import functools

import jax
import jax.numpy as jnp
from jax import lax
from jax.experimental import pallas as pl
from jax.experimental.pallas import tpu as pltpu
from jax.experimental.pallas import tpu_sc as plsc

N = 10000
N_PAD = 10112
E = 320000
D = 128
K = 4
NC = 2
NS = 16
NW = NC * NS
CH = 128
NCH_T = 80
NCH_TOT = NCH_T * NW
E_PAD = NCH_TOT * CH
ROWS_PER_TILE = N_PAD // NS
NBUF = 3
DEG_NBUF = 2
NHALF = 2
CPH = NCH_T // NHALF
DEG_NGRP = CPH // DEG_NBUF
DEG_L = 16
ZROWS = 32


@functools.lru_cache(maxsize=None)
def _sc_mesh():
    return plsc.VectorSubcoreMesh(
        core_axis_name="c", subcore_axis_name="s",
        num_cores=NC, num_subcores=NS)


def _zero_acc_slice(acc, zbuf, zr, sid):
    zero16 = jnp.zeros((16,), jnp.float32)

    def zbody(i, _):
        for j in range(D // 16):
            zbuf[i, pl.ds(j * 16, 16)] = zero16
        return 0
    lax.fori_loop(0, zr, zbody, 0)
    base = sid * ROWS_PER_TILE
    nfull, tail = divmod(ROWS_PER_TILE, zr)
    for p in range(nfull):
        pltpu.sync_copy(zbuf, acc.at[pl.ds(base + p * zr, zr)])
    if tail:
        pltpu.sync_copy(zbuf.at[pl.ds(0, tail)],
                        acc.at[pl.ds(base + nfull * zr, tail)])


def _deg_body(col2, out, acc, ones_v, cidx, zbuf, s0, s1):
    cid = lax.axis_index("c")
    sid = lax.axis_index("s")
    wid = cid * NS + sid
    ssem = (s0, s1)

    _zero_acc_slice(acc, zbuf, ZROWS, sid)

    one16 = jnp.ones((16,), jnp.float32)

    def obody(i, _):
        for j in range(D // 16):
            ones_v[i, pl.ds(j * 16, 16)] = one16
        return 0
    lax.fori_loop(0, CH, obody, 0)
    plsc.subcore_barrier()

    for h in range(NHALF):
        pltpu.sync_copy(
            col2.at[pl.ds(wid * NCH_T + h * CPH, CPH)], cidx)

        def grp(g, _):
            for b in range(DEG_NBUF):
                @pl.when(g > 0)
                def _wait(b=b, g=g):
                    pltpu.make_async_copy(
                        ones_v, acc.at[cidx.at[(g - 1) * DEG_NBUF + b]],
                        ssem[b]).wait()
                pltpu.async_copy(ones_v, acc.at[cidx.at[g * DEG_NBUF + b]],
                                 ssem[b], add=True)
            return 0
        lax.fori_loop(0, DEG_NGRP, grp, 0)
        for b in range(DEG_NBUF):
            pltpu.make_async_copy(
                ones_v, acc.at[cidx.at[(DEG_NGRP - 1) * DEG_NBUF + b]],
                ssem[b]).wait()

    plsc.subcore_barrier()
    rows = pl.ds(sid * ROWS_PER_TILE, ROWS_PER_TILE)
    pltpu.sync_copy(acc.at[rows], out.at[cid, rows])


@functools.lru_cache(maxsize=None)
def _deg_call():
  return pl.kernel(
    _deg_body,
    out_type=jax.ShapeDtypeStruct((NC, N_PAD, D), jnp.float32),
    mesh=_sc_mesh(),
    scratch_types=[
        pltpu.VMEM_SHARED((N_PAD, D), jnp.float32),
        pltpu.VMEM((CH, D), jnp.float32),
        pltpu.VMEM((CPH, CH), jnp.int32),
        pltpu.VMEM((ZROWS, D), jnp.float32),
        pltpu.SemaphoreType.DMA,
        pltpu.SemaphoreType.DMA,
    ],
  )


GS_NGRP = NCH_T // NBUF
GS_FULL = (NCH_T // NBUF) * NBUF


def _gs_body(xs, row2, col2, out, acc, ribuf, cibuf, gbuf,
             g0, g1, g2, s0, s1, s2, ir0, ir1, ir2, ic0, ic1, ic2):
    cid = lax.axis_index("c")
    sid = lax.axis_index("s")
    wid = cid * NS + sid
    gsem = (g0, g1, g2)
    ssem = (s0, s1, s2)
    irsem = (ir0, ir1, ir2)
    icsem = (ic0, ic1, ic2)
    cbase = wid * NCH_T

    _zero_acc_slice(acc, gbuf.at[0], CH, sid)
    plsc.subcore_barrier()

    for b in range(NBUF):
        pltpu.sync_copy(row2.at[cbase + b], ribuf.at[b])
        pltpu.sync_copy(col2.at[cbase + b], cibuf.at[b])
        pltpu.async_copy(xs.at[ribuf.at[b]], gbuf.at[b], gsem[b])

    def grp(g, _):
        base = g * NBUF
        for b in range(NBUF):
            i = base + b
            @pl.when(g > 0)
            def _wc(b=b):
                pltpu.make_async_copy(
                    col2.at[0], cibuf.at[b], icsem[b]).wait()
            pltpu.make_async_copy(
                xs.at[ribuf.at[b]], gbuf.at[b], gsem[b]).wait()
            pltpu.async_copy(gbuf.at[b], acc.at[cibuf.at[b]],
                             ssem[b], add=True)
            @pl.when(g + 1 < GS_NGRP)
            def _pr(b=b, i=i):
                pltpu.async_copy(row2.at[cbase + i + NBUF],
                                 ribuf.at[b], irsem[b])

        @pl.when(g + 1 < GS_NGRP)
        def _prefetch(base=base):
            for b in range(NBUF):
                i = base + b
                pltpu.make_async_copy(
                    gbuf.at[b], acc.at[cibuf.at[b]], ssem[b]).wait()
                pltpu.async_copy(col2.at[cbase + i + NBUF],
                                 cibuf.at[b], icsem[b])
                pltpu.make_async_copy(
                    row2.at[0], ribuf.at[b], irsem[b]).wait()
                pltpu.async_copy(xs.at[ribuf.at[b]], gbuf.at[b], gsem[b])
        return 0
    lax.fori_loop(0, GS_NGRP, grp, 0)
    for b in range(NBUF):
        pltpu.make_async_copy(
            gbuf.at[b], acc.at[cibuf.at[b]], ssem[b]).wait()

    for t in range(GS_FULL, NCH_T):
        b = t - GS_FULL
        pltpu.sync_copy(row2.at[cbase + t], ribuf.at[b])
        pltpu.sync_copy(col2.at[cbase + t], cibuf.at[b])
        pltpu.async_copy(xs.at[ribuf.at[b]], gbuf.at[b], gsem[b])
    for t in range(GS_FULL, NCH_T):
        b = t - GS_FULL
        pltpu.make_async_copy(
            xs.at[ribuf.at[b]], gbuf.at[b], gsem[b]).wait()
        pltpu.async_copy(gbuf.at[b], acc.at[cibuf.at[b]], ssem[b], add=True)
    for t in range(GS_FULL, NCH_T):
        b = t - GS_FULL
        pltpu.make_async_copy(
            gbuf.at[b], acc.at[cibuf.at[b]], ssem[b]).wait()

    plsc.subcore_barrier()
    rows = pl.ds(sid * ROWS_PER_TILE, ROWS_PER_TILE)
    pltpu.sync_copy(acc.at[rows], out.at[cid, rows])


@functools.lru_cache(maxsize=None)
def _gs_call():
  return pl.kernel(
    _gs_body,
    out_type=jax.ShapeDtypeStruct((NC, N_PAD, D), jnp.float32),
    mesh=_sc_mesh(),
    scratch_types=[
        pltpu.VMEM_SHARED((N_PAD, D), jnp.float32),
        pltpu.VMEM((NBUF, CH), jnp.int32),
        pltpu.VMEM((NBUF, CH), jnp.int32),
        pltpu.VMEM((NBUF, CH, D), jnp.float32),
    ] + [pltpu.SemaphoreType.DMA] * 12,
  )


BN = 1024


def _deg_col(dp_ref):
    return dp_ref[0, :, 0:1] + dp_ref[1, :, 0:1]


def _xs_body(dp_ref, x_ref, o_ref):
    deg = _deg_col(dp_ref)
    r = jnp.where(deg > 0, lax.rsqrt(deg), 0.0)
    o_ref[...] = jnp.where(deg > 0, r * x_ref[...], 0.0)


_xs_call = pl.pallas_call(
    _xs_body,
    out_shape=jax.ShapeDtypeStruct((N_PAD, D), jnp.float32),
    grid=((N_PAD + BN - 1) // BN,),
    in_specs=[
        pl.BlockSpec((NC, BN, D), lambda i: (0, i, 0)),
        pl.BlockSpec((BN, D), lambda i: (i, 0)),
    ],
    out_specs=pl.BlockSpec((BN, D), lambda i: (i, 0)),
)


def _comb_body(dp_ref, hp_ref, x_ref, e_ref, w_ref, o_ref):
    deg = _deg_col(dp_ref)
    r = jnp.where(deg > 0, lax.rsqrt(deg), 0.0)
    hi = r * (hp_ref[0] + hp_ref[1])
    xb = x_ref[...]
    hcat = jnp.concatenate([hi, xb], axis=1)
    y = jnp.dot(hcat, w_ref[...], preferred_element_type=jnp.float32)
    acc = xb
    eb = e_ref[...]
    for k in range(K):
        acc = acc + eb[:, k][:, None] * y[:, k * D:(k + 1) * D]
    o_ref[...] = acc


_comb_call = pl.pallas_call(
    _comb_body,
    out_shape=jax.ShapeDtypeStruct((N, D), jnp.float32),
    grid=((N + BN - 1) // BN,),
    in_specs=[
        pl.BlockSpec((NC, BN, D), lambda i: (0, i, 0)),
        pl.BlockSpec((NC, BN, D), lambda i: (0, i, 0)),
        pl.BlockSpec((BN, D), lambda i: (i, 0)),
        pl.BlockSpec((BN, K), lambda i: (i, 0)),
        pl.BlockSpec((2 * D, K * D), lambda i: (0, 0)),
    ],
    out_specs=pl.BlockSpec((BN, D), lambda i: (i, 0)),
)


def kernel(x, adj, e, W):
    pad_idx = N + (jnp.arange(E_PAD - E, dtype=jnp.int32) % (N_PAD - N))
    row2 = jnp.concatenate([adj[0], pad_idx]).reshape(NCH_TOT, CH)
    col2 = jnp.concatenate([adj[1], pad_idx]).reshape(NCH_TOT, CH)
    deg_part = _deg_call()(col2)
    xs = _xs_call(deg_part, x)
    hi_part = _gs_call()(xs, row2, col2)
    Wr = jnp.transpose(W, (1, 0, 2)).reshape(2 * D, K * D)
    return _comb_call(deg_part, hi_part, x, e, Wr)


# speedup vs baseline: 28.8153x; 1.2444x over previous
import functools

import jax
import jax.numpy as jnp
from jax import lax
from jax.experimental import pallas as pl
from jax.experimental.pallas import tpu as pltpu
from jax.experimental.pallas import tpu_sc as plsc

N = 10000
N_PAD = 10112
E = 320000
D = 128
K = 4
NC = 2
NS = 16
NW = NC * NS
CH = 128
NCH_T = 80
NCH_TOT = NCH_T * NW
E_PAD = NCH_TOT * CH
ROWS_PER_TILE = N_PAD // NS
NBUF = 3
EPT = E_PAD // NW
DBLK = 1024
DNB = EPT // DBLK


@functools.lru_cache(maxsize=None)
def _sc_mesh():
    return plsc.VectorSubcoreMesh(
        core_axis_name="c", subcore_axis_name="s",
        num_cores=NC, num_subcores=NS)


def _zero_acc_slice(acc, zbuf, zr, sid):
    zero16 = jnp.zeros((16,), jnp.float32)

    def zbody(i, _):
        for j in range(D // 16):
            zbuf[i, pl.ds(j * 16, 16)] = zero16
        return 0
    lax.fori_loop(0, zr, zbody, 0)
    base = sid * ROWS_PER_TILE
    nfull, tail = divmod(ROWS_PER_TILE, zr)
    for p in range(nfull):
        pltpu.sync_copy(zbuf, acc.at[pl.ds(base + p * zr, zr)])
    if tail:
        pltpu.sync_copy(zbuf.at[pl.ds(0, tail)],
                        acc.at[pl.ds(base + nfull * zr, tail)])


def _lane_shift(v, idx):
    dn = lax.GatherDimensionNumbers(offset_dims=(), collapsed_slice_dims=(0,),
                                    start_index_map=(0,))
    return lax.gather(v, idx[:, None], dn, (1,),
                      mode=lax.GatherScatterMode.PROMISE_IN_BOUNDS)


def _deg_body(colf, out, deg_loc, cbuf, i0, i1):
    cid = lax.axis_index("c")
    sid = lax.axis_index("s")
    wid = cid * NS + sid
    isem = (i0, i1)
    ebase = wid * EPT

    zero16 = jnp.zeros((16,), jnp.float32)
    iota = lax.iota(jnp.int32, 16)

    def z(i, _):
        deg_loc[pl.ds(i * 16, 16)] = zero16
        return 0
    lax.fori_loop(0, N_PAD // 16, z, 0)

    pltpu.async_copy(colf.at[pl.ds(ebase, DBLK)], cbuf.at[0], isem[0])

    def blk(h, _):
        for p in range(2):
            @pl.when(lax.rem(h, 2) == p)
            def _run(p=p):
                pltpu.make_async_copy(
                    colf.at[pl.ds(ebase, DBLK)], cbuf.at[p], isem[p]).wait()

                @pl.when(h + 1 < DNB)
                def _pf():
                    pltpu.async_copy(
                        colf.at[pl.ds(ebase + (h + 1) * DBLK, DBLK)],
                        cbuf.at[1 - p], isem[1 - p])

                def step(j, _):
                    idx16 = cbuf[p, pl.ds(j * 16, 16)]
                    sidx = lax.sort(idx16)
                    prev = _lane_shift(sidx, jnp.maximum(iota - 1, 0))
                    nxt = _lane_shift(sidx, jnp.minimum(iota + 1, 15))
                    is_start = jnp.logical_or(iota == 0, sidx != prev)
                    is_end = jnp.logical_or(iota == 15, sidx != nxt)
                    start_pos = plsc.cummax(jnp.where(is_start, iota, 0))
                    cnt = (iota - start_pos + 1).astype(jnp.float32)
                    g = plsc.load_gather(deg_loc, [sidx])
                    plsc.store_scatter(deg_loc, [sidx], g + cnt, mask=is_end)
                    return 0
                lax.fori_loop(0, DBLK // 16, step, 0)
        return 0
    lax.fori_loop(0, DNB, blk, 0)

    pltpu.sync_copy(deg_loc, out.at[wid, 0])


@functools.lru_cache(maxsize=None)
def _deg_call():
  return pl.kernel(
    _deg_body,
    out_type=jax.ShapeDtypeStruct((NW, 1, N_PAD), jnp.float32),
    mesh=_sc_mesh(),
    compiler_params=pltpu.CompilerParams(needs_layout_passes=False),
    scratch_types=[
        pltpu.VMEM((N_PAD,), jnp.float32),
        pltpu.VMEM((2, DBLK), jnp.int32),
        pltpu.SemaphoreType.DMA,
        pltpu.SemaphoreType.DMA,
    ],
  )


GS_NGRP = NCH_T // NBUF
GS_FULL = (NCH_T // NBUF) * NBUF


def _gs_body(xs, row2, col2, out, acc, ribuf, cibuf, gbuf,
             g0, g1, g2, s0, s1, s2, ir0, ir1, ir2, ic0, ic1, ic2):
    cid = lax.axis_index("c")
    sid = lax.axis_index("s")
    wid = cid * NS + sid
    gsem = (g0, g1, g2)
    ssem = (s0, s1, s2)
    irsem = (ir0, ir1, ir2)
    icsem = (ic0, ic1, ic2)
    cbase = wid * NCH_T

    _zero_acc_slice(acc, gbuf.at[0], CH, sid)
    plsc.subcore_barrier()

    for b in range(NBUF):
        pltpu.sync_copy(row2.at[cbase + b], ribuf.at[b])
        pltpu.sync_copy(col2.at[cbase + b], cibuf.at[b])
        pltpu.async_copy(xs.at[ribuf.at[b]], gbuf.at[b], gsem[b])

    def grp(g, _):
        base = g * NBUF
        for b in range(NBUF):
            i = base + b
            @pl.when(g > 0)
            def _wc(b=b):
                pltpu.make_async_copy(
                    col2.at[0], cibuf.at[b], icsem[b]).wait()
            pltpu.make_async_copy(
                xs.at[ribuf.at[b]], gbuf.at[b], gsem[b]).wait()
            pltpu.async_copy(gbuf.at[b], acc.at[cibuf.at[b]],
                             ssem[b], add=True)
            @pl.when(g + 1 < GS_NGRP)
            def _pr(b=b, i=i):
                pltpu.async_copy(row2.at[cbase + i + NBUF],
                                 ribuf.at[b], irsem[b])

        @pl.when(g + 1 < GS_NGRP)
        def _prefetch(base=base):
            for b in range(NBUF):
                i = base + b
                pltpu.make_async_copy(
                    gbuf.at[b], acc.at[cibuf.at[b]], ssem[b]).wait()
                pltpu.async_copy(col2.at[cbase + i + NBUF],
                                 cibuf.at[b], icsem[b])
                pltpu.make_async_copy(
                    row2.at[0], ribuf.at[b], irsem[b]).wait()
                pltpu.async_copy(xs.at[ribuf.at[b]], gbuf.at[b], gsem[b])
        return 0
    lax.fori_loop(0, GS_NGRP, grp, 0)
    for b in range(NBUF):
        pltpu.make_async_copy(
            gbuf.at[b], acc.at[cibuf.at[b]], ssem[b]).wait()

    for t in range(GS_FULL, NCH_T):
        b = t - GS_FULL
        pltpu.sync_copy(row2.at[cbase + t], ribuf.at[b])
        pltpu.sync_copy(col2.at[cbase + t], cibuf.at[b])
        pltpu.async_copy(xs.at[ribuf.at[b]], gbuf.at[b], gsem[b])
    for t in range(GS_FULL, NCH_T):
        b = t - GS_FULL
        pltpu.make_async_copy(
            xs.at[ribuf.at[b]], gbuf.at[b], gsem[b]).wait()
        pltpu.async_copy(gbuf.at[b], acc.at[cibuf.at[b]], ssem[b], add=True)
    for t in range(GS_FULL, NCH_T):
        b = t - GS_FULL
        pltpu.make_async_copy(
            gbuf.at[b], acc.at[cibuf.at[b]], ssem[b]).wait()

    plsc.subcore_barrier()
    rows = pl.ds(sid * ROWS_PER_TILE, ROWS_PER_TILE)
    pltpu.sync_copy(acc.at[rows], out.at[cid, rows])


@functools.lru_cache(maxsize=None)
def _gs_call():
  return pl.kernel(
    _gs_body,
    out_type=jax.ShapeDtypeStruct((NC, N_PAD, D), jnp.float32),
    mesh=_sc_mesh(),
    scratch_types=[
        pltpu.VMEM_SHARED((N_PAD, D), jnp.float32),
        pltpu.VMEM((NBUF, CH), jnp.int32),
        pltpu.VMEM((NBUF, CH), jnp.int32),
        pltpu.VMEM((NBUF, CH, D), jnp.float32),
    ] + [pltpu.SemaphoreType.DMA] * 12,
  )


BN = 1024


def _deg_col(dp_ref):
    return lax.dot_general(dp_ref[:, 0, :], jnp.ones((NW, 1), jnp.float32),
                           (((0,), (0,)), ((), ())),
                           preferred_element_type=jnp.float32)


def _xs_body(dp_ref, x_ref, o_ref):
    deg = _deg_col(dp_ref)
    r = jnp.where(deg > 0, lax.rsqrt(deg), 0.0)
    o_ref[...] = jnp.where(deg > 0, r * x_ref[...], 0.0)


_xs_call = pl.pallas_call(
    _xs_body,
    out_shape=jax.ShapeDtypeStruct((N_PAD, D), jnp.float32),
    grid=((N_PAD + BN - 1) // BN,),
    in_specs=[
        pl.BlockSpec((NW, 1, BN), lambda i: (0, 0, i)),
        pl.BlockSpec((BN, D), lambda i: (i, 0)),
    ],
    out_specs=pl.BlockSpec((BN, D), lambda i: (i, 0)),
)


def _comb_body(dp_ref, hp_ref, x_ref, e_ref, w_ref, o_ref):
    deg = _deg_col(dp_ref)
    r = jnp.where(deg > 0, lax.rsqrt(deg), 0.0)
    hi = r * (hp_ref[0] + hp_ref[1])
    xb = x_ref[...]
    hcat = jnp.concatenate([hi, xb], axis=1)
    y = jnp.dot(hcat, w_ref[...], preferred_element_type=jnp.float32)
    acc = xb
    eb = e_ref[...]
    for k in range(K):
        acc = acc + eb[:, k][:, None] * y[:, k * D:(k + 1) * D]
    o_ref[...] = acc


_comb_call = pl.pallas_call(
    _comb_body,
    out_shape=jax.ShapeDtypeStruct((N, D), jnp.float32),
    grid=((N + BN - 1) // BN,),
    in_specs=[
        pl.BlockSpec((NW, 1, BN), lambda i: (0, 0, i)),
        pl.BlockSpec((NC, BN, D), lambda i: (0, i, 0)),
        pl.BlockSpec((BN, D), lambda i: (i, 0)),
        pl.BlockSpec((BN, K), lambda i: (i, 0)),
        pl.BlockSpec((2 * D, K * D), lambda i: (0, 0)),
    ],
    out_specs=pl.BlockSpec((BN, D), lambda i: (i, 0)),
)


def kernel(x, adj, e, W):
    pad_idx = N + (jnp.arange(E_PAD - E, dtype=jnp.int32) % (N_PAD - N))
    colf = jnp.concatenate([adj[1], pad_idx])
    row2 = jnp.concatenate([adj[0], pad_idx]).reshape(NCH_TOT, CH)
    col2 = colf.reshape(NCH_TOT, CH)
    deg_part = _deg_call()(colf)
    xs = _xs_call(deg_part, x)
    hi_part = _gs_call()(xs, row2, col2)
    Wr = jnp.transpose(W, (1, 0, 2)).reshape(2 * D, K * D)
    return _comb_call(deg_part, hi_part, x, e, Wr)


# speedup vs baseline: 30.0383x; 1.0424x over previous
import functools

import jax
import jax.numpy as jnp
from jax import lax
from jax.experimental import pallas as pl
from jax.experimental.pallas import tpu as pltpu
from jax.experimental.pallas import tpu_sc as plsc

N = 10000
N_PAD = 10112
E = 320000
D = 128
K = 4
NC = 2
NS = 16
NW = NC * NS
CH = 64
NCH_T = 160
NCH_TOT = NCH_T * NW
E_PAD = NCH_TOT * CH
ROWS_PER_TILE = N_PAD // NS
NBUF = 5
EPT = E_PAD // NW
DBLK = 1024
DNB = EPT // DBLK


@functools.lru_cache(maxsize=None)
def _sc_mesh():
    return plsc.VectorSubcoreMesh(
        core_axis_name="c", subcore_axis_name="s",
        num_cores=NC, num_subcores=NS)


def _zero_acc_slice(acc, zbuf, zr, sid):
    zero16 = jnp.zeros((16,), jnp.float32)

    def zbody(i, _):
        for j in range(D // 16):
            zbuf[i, pl.ds(j * 16, 16)] = zero16
        return 0
    lax.fori_loop(0, zr, zbody, 0)
    base = sid * ROWS_PER_TILE
    nfull, tail = divmod(ROWS_PER_TILE, zr)
    for p in range(nfull):
        pltpu.sync_copy(zbuf, acc.at[pl.ds(base + p * zr, zr)])
    if tail:
        pltpu.sync_copy(zbuf.at[pl.ds(0, tail)],
                        acc.at[pl.ds(base + nfull * zr, tail)])


def _lane_shift(v, idx):
    dn = lax.GatherDimensionNumbers(offset_dims=(), collapsed_slice_dims=(0,),
                                    start_index_map=(0,))
    return lax.gather(v, idx[:, None], dn, (1,),
                      mode=lax.GatherScatterMode.PROMISE_IN_BOUNDS)


def _deg_body(colf, out, deg_loc, cbuf, i0, i1):
    cid = lax.axis_index("c")
    sid = lax.axis_index("s")
    wid = cid * NS + sid
    isem = (i0, i1)
    ebase = wid * EPT

    zero16 = jnp.zeros((16,), jnp.float32)
    iota = lax.iota(jnp.int32, 16)

    def z(i, _):
        deg_loc[pl.ds(i * 16, 16)] = zero16
        return 0
    lax.fori_loop(0, N_PAD // 16, z, 0)

    pltpu.async_copy(colf.at[pl.ds(ebase, DBLK)], cbuf.at[0], isem[0])

    def blk(h, _):
        for p in range(2):
            @pl.when(lax.rem(h, 2) == p)
            def _run(p=p):
                pltpu.make_async_copy(
                    colf.at[pl.ds(ebase, DBLK)], cbuf.at[p], isem[p]).wait()

                @pl.when(h + 1 < DNB)
                def _pf():
                    pltpu.async_copy(
                        colf.at[pl.ds(ebase + (h + 1) * DBLK, DBLK)],
                        cbuf.at[1 - p], isem[1 - p])

                def step(j, _):
                    idx16 = cbuf[p, pl.ds(j * 16, 16)]
                    sidx = lax.sort(idx16)
                    prev = _lane_shift(sidx, jnp.maximum(iota - 1, 0))
                    nxt = _lane_shift(sidx, jnp.minimum(iota + 1, 15))
                    is_start = jnp.logical_or(iota == 0, sidx != prev)
                    is_end = jnp.logical_or(iota == 15, sidx != nxt)
                    start_pos = plsc.cummax(jnp.where(is_start, iota, 0))
                    cnt = (iota - start_pos + 1).astype(jnp.float32)
                    g = plsc.load_gather(deg_loc, [sidx])
                    plsc.store_scatter(deg_loc, [sidx], g + cnt, mask=is_end)
                    return 0
                lax.fori_loop(0, DBLK // 16, step, 0)
        return 0
    lax.fori_loop(0, DNB, blk, 0)

    pltpu.sync_copy(deg_loc, out.at[wid, 0])


@functools.lru_cache(maxsize=None)
def _deg_call():
  return pl.kernel(
    _deg_body,
    out_type=jax.ShapeDtypeStruct((NW, 1, N_PAD), jnp.float32),
    mesh=_sc_mesh(),
    compiler_params=pltpu.CompilerParams(needs_layout_passes=False),
    scratch_types=[
        pltpu.VMEM((N_PAD,), jnp.float32),
        pltpu.VMEM((2, DBLK), jnp.int32),
        pltpu.SemaphoreType.DMA,
        pltpu.SemaphoreType.DMA,
    ],
  )


GS_NGRP = NCH_T // NBUF
GS_FULL = (NCH_T // NBUF) * NBUF


def _gs_body(xs, row2, col2, out, acc, ribuf, cibuf, gbuf, *sems):
    cid = lax.axis_index("c")
    sid = lax.axis_index("s")
    wid = cid * NS + sid
    gsem = sems[0:NBUF]
    ssem = sems[NBUF:2 * NBUF]
    irsem = sems[2 * NBUF:3 * NBUF]
    icsem = sems[3 * NBUF:4 * NBUF]
    cbase = wid * NCH_T

    _zero_acc_slice(acc, gbuf.at[0], CH, sid)
    plsc.subcore_barrier()

    for b in range(NBUF):
        pltpu.sync_copy(row2.at[cbase + b], ribuf.at[b])
        pltpu.sync_copy(col2.at[cbase + b], cibuf.at[b])
        pltpu.async_copy(xs.at[ribuf.at[b]], gbuf.at[b], gsem[b])

    def grp(g, _):
        base = g * NBUF
        for b in range(NBUF):
            i = base + b
            @pl.when(g > 0)
            def _wc(b=b):
                pltpu.make_async_copy(
                    col2.at[0], cibuf.at[b], icsem[b]).wait()
            pltpu.make_async_copy(
                xs.at[ribuf.at[b]], gbuf.at[b], gsem[b]).wait()
            pltpu.async_copy(gbuf.at[b], acc.at[cibuf.at[b]],
                             ssem[b], add=True)
            @pl.when(g + 1 < GS_NGRP)
            def _pr(b=b, i=i):
                pltpu.async_copy(row2.at[cbase + i + NBUF],
                                 ribuf.at[b], irsem[b])

        @pl.when(g + 1 < GS_NGRP)
        def _prefetch(base=base):
            for b in range(NBUF):
                i = base + b
                pltpu.make_async_copy(
                    gbuf.at[b], acc.at[cibuf.at[b]], ssem[b]).wait()
                pltpu.async_copy(col2.at[cbase + i + NBUF],
                                 cibuf.at[b], icsem[b])
                pltpu.make_async_copy(
                    row2.at[0], ribuf.at[b], irsem[b]).wait()
                pltpu.async_copy(xs.at[ribuf.at[b]], gbuf.at[b], gsem[b])
        return 0
    lax.fori_loop(0, GS_NGRP, grp, 0)
    for b in range(NBUF):
        pltpu.make_async_copy(
            gbuf.at[b], acc.at[cibuf.at[b]], ssem[b]).wait()

    for t in range(GS_FULL, NCH_T):
        b = t - GS_FULL
        pltpu.sync_copy(row2.at[cbase + t], ribuf.at[b])
        pltpu.sync_copy(col2.at[cbase + t], cibuf.at[b])
        pltpu.async_copy(xs.at[ribuf.at[b]], gbuf.at[b], gsem[b])
    for t in range(GS_FULL, NCH_T):
        b = t - GS_FULL
        pltpu.make_async_copy(
            xs.at[ribuf.at[b]], gbuf.at[b], gsem[b]).wait()
        pltpu.async_copy(gbuf.at[b], acc.at[cibuf.at[b]], ssem[b], add=True)
    for t in range(GS_FULL, NCH_T):
        b = t - GS_FULL
        pltpu.make_async_copy(
            gbuf.at[b], acc.at[cibuf.at[b]], ssem[b]).wait()

    plsc.subcore_barrier()
    rows = pl.ds(sid * ROWS_PER_TILE, ROWS_PER_TILE)
    pltpu.sync_copy(acc.at[rows], out.at[cid, rows])


@functools.lru_cache(maxsize=None)
def _gs_call():
  return pl.kernel(
    _gs_body,
    out_type=jax.ShapeDtypeStruct((NC, N_PAD, D), jnp.float32),
    mesh=_sc_mesh(),
    scratch_types=[
        pltpu.VMEM_SHARED((N_PAD, D), jnp.float32),
        pltpu.VMEM((NBUF, CH), jnp.int32),
        pltpu.VMEM((NBUF, CH), jnp.int32),
        pltpu.VMEM((NBUF, CH, D), jnp.float32),
    ] + [pltpu.SemaphoreType.DMA] * (4 * NBUF),
  )


BN = 1024


def _deg_col(dp_ref):
    return lax.dot_general(dp_ref[:, 0, :], jnp.ones((NW, 1), jnp.float32),
                           (((0,), (0,)), ((), ())),
                           preferred_element_type=jnp.float32)


def _xs_body(dp_ref, x_ref, o_ref):
    deg = _deg_col(dp_ref)
    r = jnp.where(deg > 0, lax.rsqrt(deg), 0.0)
    o_ref[...] = jnp.where(deg > 0, r * x_ref[...], 0.0)


_xs_call = pl.pallas_call(
    _xs_body,
    out_shape=jax.ShapeDtypeStruct((N_PAD, D), jnp.float32),
    grid=((N_PAD + BN - 1) // BN,),
    in_specs=[
        pl.BlockSpec((NW, 1, BN), lambda i: (0, 0, i)),
        pl.BlockSpec((BN, D), lambda i: (i, 0)),
    ],
    out_specs=pl.BlockSpec((BN, D), lambda i: (i, 0)),
)


def _comb_body(dp_ref, hp_ref, x_ref, e_ref, w_ref, o_ref):
    deg = _deg_col(dp_ref)
    r = jnp.where(deg > 0, lax.rsqrt(deg), 0.0)
    hi = r * (hp_ref[0] + hp_ref[1])
    xb = x_ref[...]
    hcat = jnp.concatenate([hi, xb], axis=1)
    y = jnp.dot(hcat, w_ref[...], preferred_element_type=jnp.float32)
    acc = xb
    eb = e_ref[...]
    for k in range(K):
        acc = acc + eb[:, k][:, None] * y[:, k * D:(k + 1) * D]
    o_ref[...] = acc


_comb_call = pl.pallas_call(
    _comb_body,
    out_shape=jax.ShapeDtypeStruct((N, D), jnp.float32),
    grid=((N + BN - 1) // BN,),
    in_specs=[
        pl.BlockSpec((NW, 1, BN), lambda i: (0, 0, i)),
        pl.BlockSpec((NC, BN, D), lambda i: (0, i, 0)),
        pl.BlockSpec((BN, D), lambda i: (i, 0)),
        pl.BlockSpec((BN, K), lambda i: (i, 0)),
        pl.BlockSpec((2 * D, K * D), lambda i: (0, 0)),
    ],
    out_specs=pl.BlockSpec((BN, D), lambda i: (i, 0)),
)


def kernel(x, adj, e, W):
    pad_idx = N + (jnp.arange(E_PAD - E, dtype=jnp.int32) % (N_PAD - N))
    colf = jnp.concatenate([adj[1], pad_idx])
    row2 = jnp.concatenate([adj[0], pad_idx]).reshape(NCH_TOT, CH)
    col2 = colf.reshape(NCH_TOT, CH)
    deg_part = _deg_call()(colf)
    xs = _xs_call(deg_part, x)
    hi_part = _gs_call()(xs, row2, col2)
    Wr = jnp.transpose(W, (1, 0, 2)).reshape(2 * D, K * D)
    return _comb_call(deg_part, hi_part, x, e, Wr)
